# baseline (speedup 1.0000x reference)
import functools

import jax
import jax.numpy as jnp
from jax import lax
from jax.experimental import pallas as pl
from jax.experimental.pallas import tpu as pltpu
from jax.experimental.pallas import tpu_sc as plsc

NC, NS = 2, 16
CHUNK = 128


def _mesh():
    return plsc.VectorSubcoreMesh(core_axis_name="c", subcore_axis_name="s")


def _make_sc_hist(N, E):
    per_tile = E // (NC * NS)
    nfull, tail = divmod(per_tile, CHUNK)
    rows_per_tile = N // NS

    @functools.partial(
        pl.kernel,
        out_type=jax.ShapeDtypeStruct((NC * N, 16), jnp.float32),
        mesh=_mesh(),
        compiler_params=pltpu.CompilerParams(use_tc_tiling_on_sc=False),
        scratch_types=[
            pltpu.VMEM((CHUNK,), jnp.int32),
            pltpu.VMEM((tail if tail else 8,), jnp.int32),
            pltpu.VMEM((CHUNK, 16), jnp.float32),
            pltpu.VMEM((rows_per_tile, 16), jnp.float32),
            pltpu.VMEM_SHARED((N, 16), jnp.float32),
        ],
    )
    def k(dst_hbm, zeros_hbm, out_hbm, didx, didx_t, ones, bounce, slab):
        cid = lax.axis_index("c")
        sid = lax.axis_index("s")
        wid = sid * NC + cid
        row0 = sid * rows_per_tile
        e0 = jnp.where(lax.iota(jnp.int32, 16) == 0,
                       jnp.float32(1.0), jnp.float32(0.0))

        def fill(i, c):
            ones[i, pl.ds(0, 16)] = e0
            return c
        lax.fori_loop(0, CHUNK, fill, 0)
        pltpu.sync_copy(zeros_hbm, slab.at[pl.ds(row0, rows_per_tile)])
        plsc.subcore_barrier()

        ebase = wid * per_tile

        def body(i, c):
            pltpu.sync_copy(dst_hbm.at[pl.ds(ebase + i * CHUNK, CHUNK)], didx)
            pltpu.sync_copy(ones, slab.at[didx], add=True)
            return c
        lax.fori_loop(0, nfull, body, 0)
        if tail:
            pltpu.sync_copy(dst_hbm.at[pl.ds(ebase + nfull * CHUNK, tail)], didx_t)
            pltpu.sync_copy(ones.at[pl.ds(0, tail)], slab.at[didx_t], add=True)
        plsc.subcore_barrier()
        pltpu.sync_copy(slab.at[pl.ds(row0, rows_per_tile)], bounce)
        pltpu.sync_copy(bounce, out_hbm.at[pl.ds(cid * N + row0, rows_per_tile)])

    return k


def _make_sc_segsum(N, E, W):
    per_tile = E // NS
    nfull, tail = divmod(per_tile, CHUNK)
    rows_per_tile = N // NS
    assert nfull >= 8 and (nfull - 4) % 4 == 0 and tail % 8 == 0

    NIB = 4

    @functools.partial(
        pl.kernel,
        out_type=jax.ShapeDtypeStruct((NC * N, W), jnp.float32),
        mesh=_mesh(),
        compiler_params=pltpu.CompilerParams(use_tc_tiling_on_sc=False),
        scratch_types=(
            [pltpu.VMEM((CHUNK,), jnp.int32) for _ in range(NIB)]
            + [pltpu.VMEM((CHUNK,), jnp.int32) for _ in range(NIB)]
            + [
                pltpu.VMEM((tail if tail else 8,), jnp.int32),
                pltpu.VMEM((tail if tail else 8,), jnp.int32),
                pltpu.VMEM((CHUNK, W), jnp.float32),
                pltpu.VMEM((CHUNK, W), jnp.float32),
                pltpu.VMEM((tail if tail else 8, W), jnp.float32),
                pltpu.VMEM_SHARED((N, W), jnp.float32),
            ]
            + [pltpu.SemaphoreType.DMA for _ in range(NIB + 5)]
        ),
    )
    def k(t0, t1, src_hbm, dst_hbm, zeros_hbm, out_hbm, *refs):
        sidx = refs[0:NIB]
        didx = refs[NIB:2 * NIB]
        sidx_t, didx_t, rows0, rows1, rows_t, slab = refs[2 * NIB:2 * NIB + 6]
        sems = refs[2 * NIB + 6:]
        isem = sems[0:NIB]
        gsem = sems[NIB:NIB + 2]
        ssem = sems[NIB + 2:NIB + 4]
        tsem = sems[NIB + 4]
        rows = (rows0, rows1)
        cid = lax.axis_index("c")
        sid = lax.axis_index("s")
        row0 = sid * rows_per_tile
        pltpu.sync_copy(zeros_hbm, slab.at[pl.ds(row0, rows_per_tile)])
        plsc.subcore_barrier()

        ebase = sid * per_tile

        def issue_idx(i, j):
            off = ebase + i * CHUNK
            pltpu.async_copy(src_hbm.at[pl.ds(off, CHUNK)], sidx[j], isem[j])
            pltpu.async_copy(dst_hbm.at[pl.ds(off, CHUNK)], didx[j], isem[j])

        def wait_idx(i, j):
            off = ebase + i * CHUNK
            pltpu.make_async_copy(src_hbm.at[pl.ds(off, CHUNK)], sidx[j], isem[j]).wait()
            pltpu.make_async_copy(dst_hbm.at[pl.ds(off, CHUNK)], didx[j], isem[j]).wait()

        def issue_gather(b, j):
            @pl.when(cid == 0)
            def _g0():
                pltpu.async_copy(t0.at[sidx[j]], rows[b], gsem[b])

            @pl.when(cid == 1)
            def _g1():
                pltpu.async_copy(t1.at[sidx[j]], rows[b], gsem[b])

        def wait_gather(b, j):
            @pl.when(cid == 0)
            def _w0():
                pltpu.make_async_copy(t0.at[sidx[j]], rows[b], gsem[b]).wait()

            @pl.when(cid == 1)
            def _w1():
                pltpu.make_async_copy(t1.at[sidx[j]], rows[b], gsem[b]).wait()

        def issue_scatter(b, j):
            pltpu.async_copy(rows[b], slab.at[didx[j]], ssem[b], add=True)

        def wait_scatter(b, j):
            pltpu.make_async_copy(rows[b], slab.at[didx[j]], ssem[b]).wait()

        def step(i, b, j, first=False, issue_i=True, issue_g=True):
            wait_gather(b, j)
            if not first:
                wait_scatter(1 - b, (j - 1) % NIB)
            issue_scatter(b, j)
            if issue_g:
                wait_idx(i + 1, (j + 1) % NIB)
                issue_gather(1 - b, (j + 1) % NIB)
            if issue_i:
                issue_idx(i + 3, (j + 3) % NIB)

        issue_idx(0, 0)
        issue_idx(1, 1)
        wait_idx(0, 0)
        issue_gather(0, 0)
        issue_idx(2, 2)
        step(0, 0, 0, first=True)

        def body4(s, c):
            i = 1 + s * 4
            step(i, 1, 1)
            step(i + 1, 0, 2)
            step(i + 2, 1, 3)
            step(i + 3, 0, 0)
            return c
        lax.fori_loop(0, (nfull - 4) // 4, body4, 0)
        step(nfull - 3, 1, 1, issue_i=False)
        step(nfull - 2, 0, 2, issue_i=False)
        step(nfull - 1, 1, 3, issue_i=False, issue_g=False)
        wait_scatter(1, 3)
        if tail:
            b = ebase + nfull * CHUNK
            pltpu.sync_copy(src_hbm.at[pl.ds(b, tail)], sidx_t)
            pltpu.sync_copy(dst_hbm.at[pl.ds(b, tail)], didx_t)

            @pl.when(cid == 0)
            def _t0():
                pltpu.async_copy(t0.at[sidx_t], rows_t, tsem).wait()

            @pl.when(cid == 1)
            def _t1():
                pltpu.async_copy(t1.at[sidx_t], rows_t, tsem).wait()

            pltpu.sync_copy(rows_t, slab.at[didx_t], add=True)
        plsc.subcore_barrier()
        pltpu.sync_copy(slab.at[pl.ds(row0, rows_per_tile)],
                        out_hbm.at[pl.ds(cid * N + row0, rows_per_tile)])

    return k


_BN = 2000


def _tc_pre(x, dis, gb, aux_w8):
    N, D = x.shape
    grid = (N // _BN,)

    def body(x_r, dis_r, gb_r, aw_r, ls_o, filmed_o, hp0_o, hp1_o):
        gbv = gb_r[...]
        gamma, beta = gbv[:, :D], gbv[:, D:]
        aux_w = aw_r[...]
        xv = x_r[...]
        ls_o[...] = jnp.sum(xv[:, :8] * aux_w, axis=1, keepdims=True)
        filmed = (1.0 + 0.5 * gamma) * xv + 0.3 * beta
        filmed_o[...] = filmed
        hp = filmed * dis_r[...]
        hp0_o[...] = hp[:, :D // 2]
        hp1_o[...] = hp[:, D // 2:]

    full = lambda s: pl.BlockSpec(s, lambda i: (0, 0))
    outs = pl.pallas_call(
        body,
        grid=grid,
        in_specs=[
            pl.BlockSpec((_BN, D), lambda i: (i, 0)),
            pl.BlockSpec((_BN, 1), lambda i: (i, 0)),
            full((1, 2 * D)),
            full((1, 8)),
        ],
        out_specs=[
            pl.BlockSpec((_BN, 1), lambda i: (i, 0)),
            pl.BlockSpec((_BN, D), lambda i: (i, 0)),
            pl.BlockSpec((_BN, D // 2), lambda i: (i, 0)),
            pl.BlockSpec((_BN, D // 2), lambda i: (i, 0)),
        ],
        out_shape=[
            jax.ShapeDtypeStruct((N, 1), jnp.float32),
            jax.ShapeDtypeStruct((N, D), jnp.float32),
            jax.ShapeDtypeStruct((N, D // 2), jnp.float32),
            jax.ShapeDtypeStruct((N, D // 2), jnp.float32),
        ],
    )(x, dis, gb, aux_w8)
    return outs


def _tc_layer(slab, h, dis, di, W, b, relu):
    N, K = h.shape
    H2 = W.shape[1]
    nb = N // _BN

    def body(s0_r, s1_r, h_r, dis_r, di_r, W_r, b_r, hn_o, hp0_o, hp1_o):
        dis = dis_r[...]
        agg = jnp.concatenate([s0_r[...], s1_r[...]], axis=1) * dis + h_r[...] * di_r[...]
        z = jnp.dot(agg, W_r[...], preferred_element_type=jnp.float32) + b_r[...]
        if relu:
            z = jnp.maximum(z, 0.0)
        hn_o[...] = z
        hp = z * dis
        hp0_o[...] = hp[:, :H2 // 2]
        hp1_o[...] = hp[:, H2 // 2:]

    return pl.pallas_call(
        body,
        grid=(nb,),
        in_specs=[
            pl.BlockSpec((_BN, K // 2), lambda i: (i, 0)),
            pl.BlockSpec((_BN, K // 2), lambda i, _nb=nb: (i + _nb, 0)),
            pl.BlockSpec((_BN, K), lambda i: (i, 0)),
            pl.BlockSpec((_BN, 1), lambda i: (i, 0)),
            pl.BlockSpec((_BN, 1), lambda i: (i, 0)),
            pl.BlockSpec(W.shape, lambda i: (0, 0)),
            pl.BlockSpec(b.shape, lambda i: (0, 0)),
        ],
        out_specs=[
            pl.BlockSpec((_BN, H2), lambda i: (i, 0)),
            pl.BlockSpec((_BN, H2 // 2), lambda i: (i, 0)),
            pl.BlockSpec((_BN, H2 // 2), lambda i: (i, 0)),
        ],
        out_shape=[
            jax.ShapeDtypeStruct((N, H2), jnp.float32),
            jax.ShapeDtypeStruct((N, H2 // 2), jnp.float32),
            jax.ShapeDtypeStruct((N, H2 // 2), jnp.float32),
        ],
    )(slab, slab, h, dis, di, W, b)


def _tc_final(slab, h, dis, di, ndeg, nu2, W3, b3, attW1, attb1,
              attW2, attb2, outW1, outb1, outW2, outb2):
    N, K = h.shape
    nb = N // _BN

    def body(s0_r, s1_r, h_r, dis_r, di_r, nd_r, nu_r, W3_r, b3_r, aW1_r,
             ab1_r, aW2_r, ab2_r, oW1_r, ob1_r, oW2_r, ob2_r, main_o):
        dis = dis_r[...]
        agg = jnp.concatenate([s0_r[...], s1_r[...]], axis=1) * dis + h_r[...] * di_r[...]
        h4 = jnp.dot(agg, W3_r[...], preferred_element_type=jnp.float32) + b3_r[...]
        nu_col = jnp.broadcast_to(nu_r[...], (h4.shape[0], 1))
        ai = jnp.concatenate([h4, nu_col, nd_r[...]], axis=1)
        t = jnp.dot(ai, aW1_r[...], preferred_element_type=jnp.float32) + ab1_r[...]
        t = jnp.maximum(t, 0.0)
        aw = jnp.dot(t, aW2_r[...], preferred_element_type=jnp.float32) + ab2_r[...]
        aw = 1.0 / (1.0 + jnp.exp(-aw))
        att = h4 * aw
        u = jnp.maximum(
            jnp.dot(att, oW1_r[...], preferred_element_type=jnp.float32) + ob1_r[...], 0.0)
        main_o[...] = jnp.dot(u, oW2_r[...], preferred_element_type=jnp.float32) + ob2_r[...]

    full = lambda s: pl.BlockSpec(s, lambda i: (0, 0))
    return pl.pallas_call(
        body,
        grid=(nb,),
        in_specs=[
            pl.BlockSpec((_BN, K // 2), lambda i: (i, 0)),
            pl.BlockSpec((_BN, K // 2), lambda i, _nb=nb: (i + _nb, 0)),
            pl.BlockSpec((_BN, K), lambda i: (i, 0)),
            pl.BlockSpec((_BN, 1), lambda i: (i, 0)),
            pl.BlockSpec((_BN, 1), lambda i: (i, 0)),
            pl.BlockSpec((_BN, 1), lambda i: (i, 0)),
            full((1, 1)),
            full(W3.shape), full(b3.shape),
            full(attW1.shape),
            full(attb1.shape), full(attW2.shape), full(attb2.shape),
            full(outW1.shape), full(outb1.shape), full(outW2.shape), full(outb2.shape),
        ],
        out_specs=[pl.BlockSpec((_BN, 1), lambda i: (i, 0))],
        out_shape=[jax.ShapeDtypeStruct((N, 1), jnp.float32)],
    )(slab, slab, h, dis, di, ndeg, nu2, W3, b3, attW1, attb1,
      attW2, attb2, outW1, outb1, outW2, outb2)[0]


def kernel(x, edge_index, nu, node_degrees, params):
    p = params
    N, D = x.shape
    E = edge_index.shape[1]
    H = p["gcn_W1"].shape[1]
    src = edge_index[0]
    dst = edge_index[1]
    nu2 = nu.reshape(1, 1)
    rows_per_tile = N // NS
    zeros_w = jnp.zeros((rows_per_tile, max(H // 2, 16)), jnp.float32)

    hist = _make_sc_hist(N, E)(dst, zeros_w[:, :16])

    aux_h = jax.nn.relu(nu2 @ p["aux_W1"] + p["aux_b1"])
    aux_w = jax.nn.softmax(aux_h @ p["aux_W2"] + p["aux_b2"], axis=-1)
    aux_w8 = jnp.pad(aux_w, ((0, 0), (0, 3)))
    gb = jax.nn.relu(nu2 @ p["film_W1"] + p["film_b1"]) @ p["film_W2"] + p["film_b2"]
    deg = hist[:N, 0:1] + hist[N:, 0:1] + 1.0
    dis = deg ** -0.5
    di = 1.0 / deg
    ls, filmed, hp0, hp1 = _tc_pre(x, dis, gb, aux_w8)

    s1 = _make_sc_segsum(N, E, D // 2)(hp0, hp1, src, dst, zeros_w[:, :D // 2])
    h2, h2p0, h2p1 = _tc_layer(s1, filmed, dis, di, p["gcn_W1"],
                               p["gcn_b1"].reshape(1, -1), True)
    s2 = _make_sc_segsum(N, E, H // 2)(h2p0, h2p1, src, dst, zeros_w[:, :H // 2])
    h3, h3p0, h3p1 = _tc_layer(s2, h2, dis, di, p["gcn_W2"],
                               p["gcn_b2"].reshape(1, -1), True)
    s3 = _make_sc_segsum(N, E, H // 2)(h3p0, h3p1, src, dst, zeros_w[:, :H // 2])

    main = _tc_final(
        s3, h3, dis, di, node_degrees, nu2,
        p["gcn_W3"], p["gcn_b3"].reshape(1, -1),
        p["att_W1"],
        p["att_b1"].reshape(1, -1), p["att_W2"], p["att_b2"].reshape(1, -1),
        p["out_W1"], p["out_b1"].reshape(1, -1),
        p["out_W2"], p["out_b2"].reshape(1, -1))
    return main, ls
